# baseline (speedup 1.0000x reference)
import functools

import jax
import jax.numpy as jnp
from jax import lax
from jax.experimental import pallas as pl
from jax.experimental.pallas import tpu as pltpu
from jax.experimental.pallas import tpu_sc as plsc

N_NODES = 10000
N_EDGES = 320000
D_IN = 128
D_HID = 64
D_OUT = 40
D_OUT_PAD = 48

NC = 2
NS = 16
NW = NC * NS
CHUNK = 128
EDGES_PER_W = N_EDGES // NW
NBUF = 2
N_CHUNKS = 80
EDGES_PER_W_PAD = N_CHUNKS * CHUNK
N_ACC = 10240
ROWS_PER_SUB = N_ACC // NS

_mesh = plsc.VectorSubcoreMesh(core_axis_name="c", subcore_axis_name="s")
_sc_params = pltpu.CompilerParams(use_tc_tiling_on_sc=False)


def _wid():
    return lax.axis_index("s") * NC + lax.axis_index("c")


def _deg_body(col_hbm, ones_hbm, zeros_hbm, out_hbm, col_v, ones_v, acc, ssem):
    cid = lax.axis_index("c")
    sid = lax.axis_index("s")
    base = sid * ROWS_PER_SUB
    pltpu.sync_copy(col_hbm.at[_wid()], col_v)
    pltpu.sync_copy(ones_hbm, ones_v)
    pltpu.sync_copy(zeros_hbm, acc.at[pl.ds(base, ROWS_PER_SUB)])
    plsc.subcore_barrier()

    @pl.loop(0, N_CHUNKS, step=NBUF)
    def _(j):
        descs = [
            pltpu.async_copy(ones_v, acc.at[col_v.at[j + b]], ssem.at[b],
                             add=True)
            for b in range(NBUF)
        ]
        for dsc in descs:
            dsc.wait()

    plsc.subcore_barrier()
    pltpu.sync_copy(acc.at[pl.ds(base, ROWS_PER_SUB)],
                    out_hbm.at[cid, pl.ds(base, ROWS_PER_SUB)])


def _make_deg_kernel():
    return pl.kernel(
        _deg_body,
        out_type=jax.ShapeDtypeStruct((NC, N_ACC, 16), jnp.float32),
        mesh=_mesh,
        scratch_types=[
            pltpu.VMEM((N_CHUNKS, CHUNK), jnp.int32),
            pltpu.VMEM((CHUNK, 16), jnp.float32),
            pltpu.VMEM_SHARED((N_ACC, 16), jnp.float32),
            pltpu.SemaphoreType.DMA((NBUF,)),
        ],
        compiler_params=_sc_params,
    )


def _agg_body(d, h_hbm, row_hbm, col_hbm, zeros_hbm, out_hbm,
              row_v, col_v, msg_v, acc, gsem, ssem):
    cid = lax.axis_index("c")
    sid = lax.axis_index("s")
    base = sid * ROWS_PER_SUB
    pltpu.sync_copy(row_hbm.at[_wid()], row_v)
    pltpu.sync_copy(col_hbm.at[_wid()], col_v)
    pltpu.sync_copy(zeros_hbm, acc.at[pl.ds(base, ROWS_PER_SUB)])
    plsc.subcore_barrier()

    @pl.loop(0, N_CHUNKS, step=NBUF)
    def _(j):
        gd = [
            pltpu.async_copy(h_hbm.at[row_v.at[j + b]], msg_v.at[b],
                             gsem.at[b])
            for b in range(NBUF)
        ]
        sd = []
        for b in range(NBUF):
            gd[b].wait()
            sd.append(
                pltpu.async_copy(msg_v.at[b], acc.at[col_v.at[j + b]],
                                 ssem.at[b], add=True))
        for dsc in sd:
            dsc.wait()

    plsc.subcore_barrier()
    pltpu.sync_copy(acc.at[pl.ds(base, ROWS_PER_SUB)],
                    out_hbm.at[cid, pl.ds(base, ROWS_PER_SUB)])


def _make_agg_kernel(d):
    return pl.kernel(
        functools.partial(_agg_body, d),
        out_type=jax.ShapeDtypeStruct((NC, N_ACC, d), jnp.float32),
        mesh=_mesh,
        scratch_types=[
            pltpu.VMEM((N_CHUNKS, CHUNK), jnp.int32),
            pltpu.VMEM((N_CHUNKS, CHUNK), jnp.int32),
            pltpu.VMEM((NBUF, CHUNK, d), jnp.float32),
            pltpu.VMEM_SHARED((N_ACC, d), jnp.float32),
            pltpu.SemaphoreType.DMA((NBUF,)),
            pltpu.SemaphoreType.DMA((NBUF,)),
        ],
        compiler_params=_sc_params,
    )


_BLK = 2000


def _dis_from_cnt(cnt_ref):
    cnt = cnt_ref[0, :, 0] + cnt_ref[1, :, 0]
    return lax.rsqrt(cnt + 1.0)


def _pre_body(x_ref, w_ref, cnt_ref, out_ref):
    dis = _dis_from_cnt(cnt_ref)
    h = jnp.dot(x_ref[...], w_ref[...], preferred_element_type=jnp.float32)
    out_ref[...] = h * dis[:, None]


def _mid_body(agg_ref, h1_ref, cnt_ref, b1_ref, w2_ref, out_ref):
    dis = _dis_from_cnt(cnt_ref)
    z = dis[:, None] * (agg_ref[0] + agg_ref[1] + h1_ref[...]) + b1_ref[...]
    z = jnp.maximum(z, 0.0)
    h2 = jnp.dot(z, w2_ref[...], preferred_element_type=jnp.float32)
    out_ref[...] = h2 * dis[:, None]


def _post_body(agg_ref, h2_ref, cnt_ref, b2_ref, out_ref):
    dis = _dis_from_cnt(cnt_ref)
    z = dis[:, None] * (agg_ref[0] + agg_ref[1] + h2_ref[...]) + b2_ref[...]
    z = z[:, :D_OUT]
    m = jnp.max(z, axis=1, keepdims=True)
    e = jnp.exp(z - m)
    out_ref[...] = (z - m) - jnp.log(jnp.sum(e, axis=1, keepdims=True))


def _row_spec(d):
    return pl.BlockSpec((_BLK, d), lambda i: (i, 0))


def _full_spec(shape):
    return pl.BlockSpec(shape, lambda i: tuple(0 for _ in shape))


def _acc_spec(d):
    return pl.BlockSpec((NC, _BLK, d), lambda i: (0, i, 0))


def kernel(x, edge_index, W1, b1, W2, b2):
    ei = edge_index.astype(jnp.int32)
    pad = EDGES_PER_W_PAD * NW - N_EDGES
    row = jnp.concatenate([ei[0], jnp.zeros((pad,), jnp.int32)])
    col = jnp.concatenate([ei[1], jnp.full((pad,), N_NODES, jnp.int32)])
    row3 = row.reshape(NW, N_CHUNKS, CHUNK)
    col3 = col.reshape(NW, N_CHUNKS, CHUNK)

    ones16 = jnp.ones((CHUNK, 16), jnp.float32)
    zeros16 = jnp.zeros((ROWS_PER_SUB, 16), jnp.float32)
    zeros_hid = jnp.zeros((ROWS_PER_SUB, D_HID), jnp.float32)
    zeros_out = jnp.zeros((ROWS_PER_SUB, D_OUT_PAD), jnp.float32)

    cnt = _make_deg_kernel()(col3, ones16, zeros16)

    W2p = jnp.zeros((D_HID, D_OUT_PAD), jnp.float32).at[:, :D_OUT].set(W2)
    b2p = jnp.zeros((1, D_OUT_PAD), jnp.float32).at[0, :D_OUT].set(b2)
    b1r = b1.reshape(1, D_HID)

    grid = N_NODES // _BLK

    h1 = pl.pallas_call(
        _pre_body,
        grid=(grid,),
        in_specs=[_row_spec(D_IN), _full_spec((D_IN, D_HID)), _acc_spec(16)],
        out_specs=_row_spec(D_HID),
        out_shape=jax.ShapeDtypeStruct((N_NODES, D_HID), jnp.float32),
    )(x, W1, cnt)

    agg1 = _make_agg_kernel(D_HID)(h1, row3, col3, zeros_hid)

    h2 = pl.pallas_call(
        _mid_body,
        grid=(grid,),
        in_specs=[_acc_spec(D_HID), _row_spec(D_HID), _acc_spec(16),
                  _full_spec((1, D_HID)), _full_spec((D_HID, D_OUT_PAD))],
        out_specs=_row_spec(D_OUT_PAD),
        out_shape=jax.ShapeDtypeStruct((N_NODES, D_OUT_PAD), jnp.float32),
    )(agg1, h1, cnt, b1r, W2p)

    agg2 = _make_agg_kernel(D_OUT_PAD)(h2, row3, col3, zeros_out)

    out = pl.pallas_call(
        _post_body,
        grid=(grid,),
        in_specs=[_acc_spec(D_OUT_PAD), _row_spec(D_OUT_PAD), _acc_spec(16),
                  _full_spec((1, D_OUT_PAD))],
        out_specs=_row_spec(D_OUT),
        out_shape=jax.ShapeDtypeStruct((N_NODES, D_OUT), jnp.float32),
    )(agg2, h2, cnt, b2p)

    return out


# speedup vs baseline: 1.9726x; 1.9726x over previous
import functools

import jax
import jax.numpy as jnp
from jax import lax
from jax.experimental import pallas as pl
from jax.experimental.pallas import tpu as pltpu
from jax.experimental.pallas import tpu_sc as plsc

N_NODES = 10000
N_EDGES = 320000
D_IN = 128
D_HID = 64
D_OUT = 40
D_OUT_PAD = 48

NC = 2
NS = 16
NW = NC * NS
CHUNK = 128
EDGES_PER_W = N_EDGES // NW
N_CHUNKS = 79
EDGES_PER_W_PAD = N_CHUNKS * CHUNK
TBL_PER_SUB = N_NODES // NS
N_ACC = 10240
ROWS_PER_SUB = N_ACC // NS

_mesh = plsc.VectorSubcoreMesh(core_axis_name="c", subcore_axis_name="s")
_sc_params = pltpu.CompilerParams(use_tc_tiling_on_sc=False)


def _wid():
    return lax.axis_index("s") * NC + lax.axis_index("c")


def _deg_body(col_hbm, ones_hbm, zeros_hbm, out_hbm, col_v, ones_v, acc):
    cid = lax.axis_index("c")
    sid = lax.axis_index("s")
    base = sid * ROWS_PER_SUB
    pltpu.sync_copy(col_hbm.at[_wid()], col_v)
    pltpu.sync_copy(ones_hbm, ones_v)
    pltpu.sync_copy(zeros_hbm, acc.at[pl.ds(base, ROWS_PER_SUB)])
    plsc.subcore_barrier()

    @pl.loop(0, N_CHUNKS)
    def _(j):
        pltpu.sync_copy(ones_v, acc.at[col_v.at[j]], add=True)

    plsc.subcore_barrier()
    pltpu.sync_copy(acc.at[pl.ds(base, ROWS_PER_SUB)],
                    out_hbm.at[cid, pl.ds(base, ROWS_PER_SUB)])


def _make_deg_kernel():
    return pl.kernel(
        _deg_body,
        out_type=jax.ShapeDtypeStruct((NC, N_ACC, 16), jnp.float32),
        mesh=_mesh,
        scratch_types=[
            pltpu.VMEM((N_CHUNKS, CHUNK), jnp.int32),
            pltpu.VMEM((CHUNK, 16), jnp.float32),
            pltpu.VMEM_SHARED((N_ACC, 16), jnp.float32),
        ],
        compiler_params=_sc_params,
    )


def _agg_body(d, h_hbm, row_hbm, col_hbm, zeros_hbm, out_hbm,
              row_v, col_v, msg_v, acc, tbl):
    cid = lax.axis_index("c")
    sid = lax.axis_index("s")
    base = sid * ROWS_PER_SUB
    pltpu.sync_copy(row_hbm.at[_wid()], row_v)
    pltpu.sync_copy(col_hbm.at[_wid()], col_v)
    pltpu.sync_copy(zeros_hbm, acc.at[pl.ds(base, ROWS_PER_SUB)])
    tbase = sid * TBL_PER_SUB
    pltpu.sync_copy(h_hbm.at[pl.ds(tbase, TBL_PER_SUB)],
                    tbl.at[pl.ds(tbase, TBL_PER_SUB)])
    plsc.subcore_barrier()

    @pl.loop(0, N_CHUNKS)
    def _(j):
        pltpu.sync_copy(tbl.at[row_v.at[j]], msg_v)
        pltpu.sync_copy(msg_v, acc.at[col_v.at[j]], add=True)

    plsc.subcore_barrier()
    pltpu.sync_copy(acc.at[pl.ds(base, ROWS_PER_SUB)],
                    out_hbm.at[cid, pl.ds(base, ROWS_PER_SUB)])


def _make_agg_kernel(d):
    return pl.kernel(
        functools.partial(_agg_body, d),
        out_type=jax.ShapeDtypeStruct((NC, N_ACC, d), jnp.float32),
        mesh=_mesh,
        scratch_types=[
            pltpu.VMEM((N_CHUNKS, CHUNK), jnp.int32),
            pltpu.VMEM((N_CHUNKS, CHUNK), jnp.int32),
            pltpu.VMEM((CHUNK, d), jnp.float32),
            pltpu.VMEM_SHARED((N_ACC, d), jnp.float32),
            pltpu.VMEM_SHARED((N_NODES, d), jnp.float32),
        ],
        compiler_params=_sc_params,
    )


_BLK = 2000


def _dis_from_cnt(cnt_ref):
    cnt = cnt_ref[0, :, 0] + cnt_ref[1, :, 0]
    return lax.rsqrt(cnt + 1.0)


def _pre_body(x_ref, w_ref, cnt_ref, out_ref):
    dis = _dis_from_cnt(cnt_ref)
    h = jnp.dot(x_ref[...], w_ref[...], preferred_element_type=jnp.float32)
    out_ref[...] = h * dis[:, None]


def _mid_body(agg_ref, h1_ref, cnt_ref, b1_ref, w2_ref, out_ref):
    dis = _dis_from_cnt(cnt_ref)
    z = dis[:, None] * (agg_ref[0] + agg_ref[1] + h1_ref[...]) + b1_ref[...]
    z = jnp.maximum(z, 0.0)
    h2 = jnp.dot(z, w2_ref[...], preferred_element_type=jnp.float32)
    out_ref[...] = h2 * dis[:, None]


def _post_body(agg_ref, h2_ref, cnt_ref, b2_ref, out_ref):
    dis = _dis_from_cnt(cnt_ref)
    z = dis[:, None] * (agg_ref[0] + agg_ref[1] + h2_ref[...]) + b2_ref[...]
    z = z[:, :D_OUT]
    m = jnp.max(z, axis=1, keepdims=True)
    e = jnp.exp(z - m)
    out_ref[...] = (z - m) - jnp.log(jnp.sum(e, axis=1, keepdims=True))


def _row_spec(d):
    return pl.BlockSpec((_BLK, d), lambda i: (i, 0))


def _full_spec(shape):
    return pl.BlockSpec(shape, lambda i: tuple(0 for _ in shape))


def _acc_spec(d):
    return pl.BlockSpec((NC, _BLK, d), lambda i: (0, i, 0))


def kernel(x, edge_index, W1, b1, W2, b2):
    ei = edge_index.astype(jnp.int32)
    pad = EDGES_PER_W_PAD * NW - N_EDGES
    row = jnp.concatenate([ei[0], jnp.zeros((pad,), jnp.int32)])
    col = jnp.concatenate([ei[1], jnp.full((pad,), N_NODES, jnp.int32)])
    row3 = row.reshape(NW, N_CHUNKS, CHUNK)
    col3 = col.reshape(NW, N_CHUNKS, CHUNK)

    ones16 = jnp.ones((CHUNK, 16), jnp.float32)
    zeros16 = jnp.zeros((ROWS_PER_SUB, 16), jnp.float32)
    zeros_hid = jnp.zeros((ROWS_PER_SUB, D_HID), jnp.float32)
    zeros_out = jnp.zeros((ROWS_PER_SUB, D_OUT_PAD), jnp.float32)

    cnt = _make_deg_kernel()(col3, ones16, zeros16)

    W2p = jnp.zeros((D_HID, D_OUT_PAD), jnp.float32).at[:, :D_OUT].set(W2)
    b2p = jnp.zeros((1, D_OUT_PAD), jnp.float32).at[0, :D_OUT].set(b2)
    b1r = b1.reshape(1, D_HID)

    grid = N_NODES // _BLK

    h1 = pl.pallas_call(
        _pre_body,
        grid=(grid,),
        in_specs=[_row_spec(D_IN), _full_spec((D_IN, D_HID)), _acc_spec(16)],
        out_specs=_row_spec(D_HID),
        out_shape=jax.ShapeDtypeStruct((N_NODES, D_HID), jnp.float32),
    )(x, W1, cnt)

    agg1 = _make_agg_kernel(D_HID)(h1, row3, col3, zeros_hid)

    h2 = pl.pallas_call(
        _mid_body,
        grid=(grid,),
        in_specs=[_acc_spec(D_HID), _row_spec(D_HID), _acc_spec(16),
                  _full_spec((1, D_HID)), _full_spec((D_HID, D_OUT_PAD))],
        out_specs=_row_spec(D_OUT_PAD),
        out_shape=jax.ShapeDtypeStruct((N_NODES, D_OUT_PAD), jnp.float32),
    )(agg1, h1, cnt, b1r, W2p)

    agg2 = _make_agg_kernel(D_OUT_PAD)(h2, row3, col3, zeros_out)

    out = pl.pallas_call(
        _post_body,
        grid=(grid,),
        in_specs=[_acc_spec(D_OUT_PAD), _row_spec(D_OUT_PAD), _acc_spec(16),
                  _full_spec((1, D_OUT_PAD))],
        out_specs=_row_spec(D_OUT),
        out_shape=jax.ShapeDtypeStruct((N_NODES, D_OUT), jnp.float32),
    )(agg2, h2, cnt, b2p)

    return out


# speedup vs baseline: 2.3198x; 1.1760x over previous
import functools

import jax
import jax.numpy as jnp
from jax import lax
from jax.experimental import pallas as pl
from jax.experimental.pallas import tpu as pltpu
from jax.experimental.pallas import tpu_sc as plsc

N_NODES = 10000
N_EDGES = 320000
D_IN = 128
D_HID = 64
D_OUT = 40
D_OUT_PAD = 48

NC = 2
NS = 16
NW = NC * NS
CHUNK = 128
EDGES_PER_W = N_EDGES // NW
N_CHUNKS = 80
EDGES_PER_W_PAD = N_CHUNKS * CHUNK
TBL_PER_SUB = N_NODES // NS
N_ACC = 10240
ROWS_PER_SUB = N_ACC // NS

_mesh = plsc.VectorSubcoreMesh(core_axis_name="c", subcore_axis_name="s")
_sc_params = pltpu.CompilerParams(use_tc_tiling_on_sc=False)


def _wid():
    return lax.axis_index("s") * NC + lax.axis_index("c")


def _deg_body(col_hbm, ones_hbm, zeros_hbm, out_hbm, col_v, ones_v, acc):
    cid = lax.axis_index("c")
    sid = lax.axis_index("s")
    base = sid * ROWS_PER_SUB
    pltpu.sync_copy(col_hbm.at[_wid()], col_v)
    pltpu.sync_copy(ones_hbm, ones_v)
    pltpu.sync_copy(zeros_hbm, acc.at[pl.ds(base, ROWS_PER_SUB)])
    plsc.subcore_barrier()

    @pl.loop(0, N_CHUNKS)
    def _(j):
        pltpu.sync_copy(ones_v, acc.at[col_v.at[j]], add=True)

    plsc.subcore_barrier()
    pltpu.sync_copy(acc.at[pl.ds(base, ROWS_PER_SUB)],
                    out_hbm.at[cid, pl.ds(base, ROWS_PER_SUB)])


def _make_deg_kernel():
    return pl.kernel(
        _deg_body,
        out_type=jax.ShapeDtypeStruct((NC, N_ACC, 16), jnp.float32),
        mesh=_mesh,
        scratch_types=[
            pltpu.VMEM((N_CHUNKS, CHUNK), jnp.int32),
            pltpu.VMEM((CHUNK, 16), jnp.float32),
            pltpu.VMEM_SHARED((N_ACC, 16), jnp.float32),
        ],
        compiler_params=_sc_params,
    )


def _agg_body(d, h_hbm, row_hbm, col_hbm, zeros_hbm, out_hbm,
              row_v, col_v, msg_v, acc, tbl, gsem):
    cid = lax.axis_index("c")
    sid = lax.axis_index("s")
    base = sid * ROWS_PER_SUB
    pltpu.sync_copy(row_hbm.at[_wid()], row_v)
    pltpu.sync_copy(col_hbm.at[_wid()], col_v)
    pltpu.sync_copy(zeros_hbm, acc.at[pl.ds(base, ROWS_PER_SUB)])
    tbase = sid * TBL_PER_SUB
    pltpu.sync_copy(h_hbm.at[pl.ds(tbase, TBL_PER_SUB)],
                    tbl.at[pl.ds(tbase, TBL_PER_SUB)])
    plsc.subcore_barrier()

    for b in range(2):
        pltpu.async_copy(tbl.at[row_v.at[b]], msg_v.at[b], gsem.at[b])

    @pl.loop(0, N_CHUNKS, step=2)
    def _(j):
        for b in range(2):
            c = j + b
            pltpu.make_async_copy(tbl.at[row_v.at[c]], msg_v.at[b],
                                  gsem.at[b]).wait()
            pltpu.sync_copy(msg_v.at[b], acc.at[col_v.at[c]], add=True)

            @pl.when(c + 2 < N_CHUNKS)
            def _():
                pltpu.async_copy(tbl.at[row_v.at[c + 2]], msg_v.at[b],
                                 gsem.at[b])

    plsc.subcore_barrier()
    pltpu.sync_copy(acc.at[pl.ds(base, ROWS_PER_SUB)],
                    out_hbm.at[cid, pl.ds(base, ROWS_PER_SUB)])


def _make_agg_kernel(d):
    return pl.kernel(
        functools.partial(_agg_body, d),
        out_type=jax.ShapeDtypeStruct((NC, N_ACC, d), jnp.float32),
        mesh=_mesh,
        scratch_types=[
            pltpu.VMEM((N_CHUNKS, CHUNK), jnp.int32),
            pltpu.VMEM((N_CHUNKS, CHUNK), jnp.int32),
            pltpu.VMEM((2, CHUNK, d), jnp.float32),
            pltpu.VMEM_SHARED((N_ACC, d), jnp.float32),
            pltpu.VMEM_SHARED((N_NODES, d), jnp.float32),
            pltpu.SemaphoreType.DMA((2,)),
        ],
        compiler_params=_sc_params,
    )


_BLK = 2000


def _dis_from_cnt(cnt_ref):
    cnt = cnt_ref[0, :, 0] + cnt_ref[1, :, 0]
    return lax.rsqrt(cnt + 1.0)


def _pre_body(x_ref, w_ref, cnt_ref, out_ref):
    dis = _dis_from_cnt(cnt_ref)
    h = jnp.dot(x_ref[...], w_ref[...], preferred_element_type=jnp.float32)
    out_ref[...] = h * dis[:, None]


def _mid_body(agg_ref, h1_ref, cnt_ref, b1_ref, w2_ref, out_ref):
    dis = _dis_from_cnt(cnt_ref)
    z = dis[:, None] * (agg_ref[0] + agg_ref[1] + h1_ref[...]) + b1_ref[...]
    z = jnp.maximum(z, 0.0)
    h2 = jnp.dot(z, w2_ref[...], preferred_element_type=jnp.float32)
    out_ref[...] = h2 * dis[:, None]


def _post_body(agg_ref, h2_ref, cnt_ref, b2_ref, out_ref):
    dis = _dis_from_cnt(cnt_ref)
    z = dis[:, None] * (agg_ref[0] + agg_ref[1] + h2_ref[...]) + b2_ref[...]
    z = z[:, :D_OUT]
    m = jnp.max(z, axis=1, keepdims=True)
    e = jnp.exp(z - m)
    out_ref[...] = (z - m) - jnp.log(jnp.sum(e, axis=1, keepdims=True))


def _row_spec(d):
    return pl.BlockSpec((_BLK, d), lambda i: (i, 0))


def _full_spec(shape):
    return pl.BlockSpec(shape, lambda i: tuple(0 for _ in shape))


def _acc_spec(d):
    return pl.BlockSpec((NC, _BLK, d), lambda i: (0, i, 0))


def kernel(x, edge_index, W1, b1, W2, b2):
    ei = edge_index.astype(jnp.int32)
    pad = EDGES_PER_W_PAD * NW - N_EDGES
    row = jnp.concatenate([ei[0], jnp.zeros((pad,), jnp.int32)])
    col = jnp.concatenate([ei[1], jnp.full((pad,), N_NODES, jnp.int32)])
    row3 = row.reshape(NW, N_CHUNKS, CHUNK)
    col3 = col.reshape(NW, N_CHUNKS, CHUNK)

    ones16 = jnp.ones((CHUNK, 16), jnp.float32)
    zeros16 = jnp.zeros((ROWS_PER_SUB, 16), jnp.float32)
    zeros_hid = jnp.zeros((ROWS_PER_SUB, D_HID), jnp.float32)
    zeros_out = jnp.zeros((ROWS_PER_SUB, D_OUT_PAD), jnp.float32)

    cnt = _make_deg_kernel()(col3, ones16, zeros16)

    W2p = jnp.zeros((D_HID, D_OUT_PAD), jnp.float32).at[:, :D_OUT].set(W2)
    b2p = jnp.zeros((1, D_OUT_PAD), jnp.float32).at[0, :D_OUT].set(b2)
    b1r = b1.reshape(1, D_HID)

    grid = N_NODES // _BLK

    h1 = pl.pallas_call(
        _pre_body,
        grid=(grid,),
        in_specs=[_row_spec(D_IN), _full_spec((D_IN, D_HID)), _acc_spec(16)],
        out_specs=_row_spec(D_HID),
        out_shape=jax.ShapeDtypeStruct((N_NODES, D_HID), jnp.float32),
    )(x, W1, cnt)

    agg1 = _make_agg_kernel(D_HID)(h1, row3, col3, zeros_hid)

    h2 = pl.pallas_call(
        _mid_body,
        grid=(grid,),
        in_specs=[_acc_spec(D_HID), _row_spec(D_HID), _acc_spec(16),
                  _full_spec((1, D_HID)), _full_spec((D_HID, D_OUT_PAD))],
        out_specs=_row_spec(D_OUT_PAD),
        out_shape=jax.ShapeDtypeStruct((N_NODES, D_OUT_PAD), jnp.float32),
    )(agg1, h1, cnt, b1r, W2p)

    agg2 = _make_agg_kernel(D_OUT_PAD)(h2, row3, col3, zeros_out)

    out = pl.pallas_call(
        _post_body,
        grid=(grid,),
        in_specs=[_acc_spec(D_OUT_PAD), _row_spec(D_OUT_PAD), _acc_spec(16),
                  _full_spec((1, D_OUT_PAD))],
        out_specs=_row_spec(D_OUT),
        out_shape=jax.ShapeDtypeStruct((N_NODES, D_OUT), jnp.float32),
    )(agg2, h2, cnt, b2p)

    return out
